# Initial kernel scaffold; baseline (speedup 1.0000x reference)
import jax
import jax.numpy as jnp
from jax.experimental import pallas as pl


def kernel(x, gate_w, gate_b, w_gate_up, b_gate_up, w_down, b_down, attn_metadata):
    raise NotImplementedError("write your pallas kernel here")


# speedup vs baseline: 1.3674x; 1.3674x over previous
import functools

import jax
import jax.numpy as jnp
from jax.experimental import pallas as pl
from jax.experimental.pallas import tpu as pltpu

T = 2048
D = 1024
F = 1024
E = 8
ALPHA = 1.702
BETA = 1.0
LIMIT = 7.0

TM = 256


def _moe_body(x_ref, gw_ref, gb_ref, wgu_ref, bgu_ref, wd_ref, bd_ref,
              out_ref, acc_ref):
    e = pl.program_id(0)

    x = x_ref[...]

    logits = jax.lax.dot_general(
        x, gw_ref[...], (((1,), (1,)), ((), ())),
        preferred_element_type=jnp.float32) + gb_ref[...]
    iota_e = jax.lax.broadcasted_iota(jnp.int32, logits.shape, 1)
    v1 = jnp.max(logits, axis=1, keepdims=True)
    i1 = jnp.argmax(logits, axis=1)[:, None]
    masked = jnp.where(iota_e == i1, -jnp.inf, logits)
    v2 = jnp.max(masked, axis=1, keepdims=True)
    i2 = jnp.argmax(masked, axis=1)[:, None]
    w1 = 1.0 / (1.0 + jnp.exp(v2 - v1))
    w2 = 1.0 - w1
    route_e = jnp.where(i1 == e, w1, 0.0) + jnp.where(i2 == e, w2, 0.0)

    h = jax.lax.dot_general(
        x, wgu_ref[0], (((1,), (1,)), ((), ())),
        preferred_element_type=jnp.float32) + bgu_ref[0]
    x_glu = jnp.minimum(h[:, :F], LIMIT)
    x_lin = jnp.clip(h[:, F:], -LIMIT, LIMIT)
    act = x_glu * jax.nn.sigmoid(ALPHA * x_glu) * (x_lin + BETA)
    y = jax.lax.dot_general(
        act, wd_ref[0], (((1,), (1,)), ((), ())),
        preferred_element_type=jnp.float32) + bd_ref[0]

    contrib = route_e * y
    t = pl.program_id(1)
    rows = pl.ds(t * TM, TM)

    @pl.when(e == 0)
    def _init():
        acc_ref[rows, :] = contrib

    @pl.when(e > 0)
    def _accum():
        acc_ref[rows, :] = acc_ref[rows, :] + contrib

    @pl.when(e == E - 1)
    def _emit():
        out_ref[...] = acc_ref[rows, :]


@functools.partial(jax.jit, static_argnames=())
def _moe(x, gate_w, gate_b, w_gate_up, b_gate_up, w_down, b_down):
    grid = (E, T // TM)
    return pl.pallas_call(
        _moe_body,
        grid=grid,
        in_specs=[
            pl.BlockSpec((TM, D), lambda e, t: (t, 0)),
            pl.BlockSpec((E, D), lambda e, t: (0, 0)),
            pl.BlockSpec((1, E), lambda e, t: (0, 0)),
            pl.BlockSpec((1, 2 * F, D), lambda e, t: (e, 0, 0)),
            pl.BlockSpec((1, 1, 2 * F), lambda e, t: (e, 0, 0)),
            pl.BlockSpec((1, D, F), lambda e, t: (e, 0, 0)),
            pl.BlockSpec((1, 1, D), lambda e, t: (e, 0, 0)),
        ],
        out_specs=pl.BlockSpec((TM, D), lambda e, t: (t, 0)),
        out_shape=jax.ShapeDtypeStruct((T, D), jnp.float32),
        scratch_shapes=[pltpu.VMEM((T, D), jnp.float32)],
        compiler_params=pltpu.CompilerParams(
            dimension_semantics=("arbitrary", "arbitrary"),
        ),
    )(x, gate_w, gate_b.reshape(1, E), w_gate_up,
      b_gate_up.reshape(E, 1, 2 * F), w_down, b_down.reshape(E, 1, D))


def kernel(x, gate_w, gate_b, w_gate_up, b_gate_up, w_down, b_down,
           attn_metadata=0):
    return _moe(x, gate_w, gate_b, w_gate_up, b_gate_up, w_down, b_down)
